# baseline (speedup 1.0000x reference)
import jax
import jax.numpy as jnp
from jax.experimental import pallas as pl
from jax.experimental.pallas import tpu as pltpu

NEG_INF = float("-inf")
BLK = 32
S = 64


def _nms_body(x_ref, o_ref):
    h, w = x_ref.shape[1], x_ref.shape[2]

    def one_plane(p, carry):
        for s in range(h // S):
            r0 = s * S
            a = max(r0 - 1, 0)
            b = min(r0 + S + 1, h)
            xe = x_ref[p, a:b, :]
            col = jax.lax.broadcasted_iota(jnp.int32, xe.shape, 1)
            left = jnp.where(col == 0, NEG_INF, pltpu.roll(xe, 1, 1))
            right = jnp.where(col == w - 1, NEG_INF, pltpu.roll(xe, w - 1, 1))
            rm = jnp.maximum(jnp.maximum(left, xe), right)
            rme = jnp.concatenate([rm[:1], rm, rm[-1:]], axis=0)
            off = r0 - a
            hmax = jnp.maximum(
                jnp.maximum(rme[off : off + S], rme[off + 1 : off + 1 + S]),
                rme[off + 2 : off + 2 + S],
            )
            xc = xe[off : off + S]
            o_ref[p, r0 : r0 + S, :] = jnp.where(hmax == xc, xc, 0.0)
        return carry

    jax.lax.fori_loop(0, BLK, one_plane, 0, unroll=False)


def kernel(points):
    n, c, h, w = points.shape
    x = points.reshape(n * c, h, w)
    out = pl.pallas_call(
        _nms_body,
        grid=((n * c) // BLK,),
        in_specs=[pl.BlockSpec((BLK, h, w), lambda i: (i, 0, 0))],
        out_specs=pl.BlockSpec((BLK, h, w), lambda i: (i, 0, 0)),
        out_shape=jax.ShapeDtypeStruct((n * c, h, w), points.dtype),
    )(x)
    return out.reshape(n, c, h, w)


# speedup vs baseline: 1.4027x; 1.4027x over previous
import jax
import jax.numpy as jnp
from jax.experimental import pallas as pl

NEG_INF = float("-inf")
BLK = 32


def _nms_body(x_ref, o_ref):
    x = x_ref[...]
    left = jnp.concatenate([jnp.full_like(x[:, :, :1], NEG_INF), x[:, :, :-1]], axis=2)
    right = jnp.concatenate([x[:, :, 1:], jnp.full_like(x[:, :, :1], NEG_INF)], axis=2)
    rowmax = jnp.maximum(jnp.maximum(left, x), right)
    up = jnp.concatenate([jnp.full_like(rowmax[:, :1, :], NEG_INF), rowmax[:, :-1, :]], axis=1)
    down = jnp.concatenate([rowmax[:, 1:, :], jnp.full_like(rowmax[:, :1, :], NEG_INF)], axis=1)
    hmax = jnp.maximum(jnp.maximum(up, rowmax), down)
    o_ref[...] = jnp.where(hmax == x, x, 0.0)


def kernel(points):
    n, c, h, w = points.shape
    x = points.reshape(n * c, h, w)
    out = pl.pallas_call(
        _nms_body,
        grid=((n * c) // BLK,),
        in_specs=[pl.BlockSpec((BLK, h, w), lambda i: (i, 0, 0))],
        out_specs=pl.BlockSpec((BLK, h, w), lambda i: (i, 0, 0)),
        out_shape=jax.ShapeDtypeStruct((n * c, h, w), points.dtype),
    )(x)
    return out.reshape(n, c, h, w)
